# baseline (speedup 1.0000x reference)
import jax
import jax.numpy as jnp
from jax import lax
from jax.experimental import pallas as pl
from jax.experimental.pallas import tpu as pltpu
from jax.experimental.pallas import tpu_sc as plsc

N = 10000
E = 320000
D = 128

NC = 2
NS = 16
NW = NC * NS
LANES = 16

CHUNK = 128
NCHUNK = E // CHUNK
BASE = NCHUNK // NW
REM = NCHUNK % NW
RPT = 640
NPAD = NS * RPT


def _worker_span():
    cid = lax.axis_index("c")
    sid = lax.axis_index("s")
    wid = sid * NC + cid
    nw = BASE + jnp.where(wid < REM, 1, 0)
    a0 = wid * BASE + jnp.minimum(wid, REM)
    return cid, sid, a0, nw


def _unpack_lane(ed_v, src_row, dst_v):
    for j in range(CHUNK // LANES):
        dst_v[pl.ds(j * LANES, LANES)] = ed_v[src_row, pl.ds(j * LANES, LANES)]


def _deg_body(ed_hbm, pd_hbm, edA, edB, col_v, ew_v, zb_v, deg_s, siA, siB):
    cid, sid, a0, nw = _worker_span()

    @pl.loop(0, RPT // LANES)
    def _(i):
        zb_v[pl.ds(i * LANES, LANES)] = jnp.zeros((LANES,), jnp.float32)

    pltpu.sync_copy(zb_v, deg_s.at[pl.ds(sid * RPT, RPT)])
    plsc.subcore_barrier()

    def _scatter(ed_v):
        _unpack_lane(ed_v, 2, col_v)
        for j in range(CHUNK // LANES):
            ew_v[pl.ds(j * LANES, LANES)] = plsc.bitcast(
                ed_v[1, pl.ds(j * LANES, LANES)], jnp.float32)
        pltpu.sync_copy(ew_v, deg_s.at[col_v], add=True)

    pltpu.sync_copy(ed_hbm.at[a0], edA)

    @pl.when(nw > 1)
    def _():
        pltpu.async_copy(ed_hbm.at[a0 + 1], edB, siB)

    @pl.loop(0, (nw + 1) // 2)
    def _(p):
        i = 2 * p

        @pl.when(i > 0)
        def _():
            pltpu.make_async_copy(ed_hbm.at[0], edA, siA).wait()

        _scatter(edA)

        @pl.when(i + 2 < nw)
        def _():
            pltpu.async_copy(ed_hbm.at[a0 + i + 2], edA, siA)

        @pl.when(i + 1 < nw)
        def _():
            pltpu.make_async_copy(ed_hbm.at[0], edB, siB).wait()
            _scatter(edB)

            @pl.when(i + 3 < nw)
            def _():
                pltpu.async_copy(ed_hbm.at[a0 + i + 3], edB, siB)

    plsc.subcore_barrier()
    pltpu.sync_copy(deg_s.at[pl.ds(sid * RPT, RPT)],
                    pd_hbm.at[cid, pl.ds(sid * RPT, RPT)])


def _deg_partials(ed):
    return pl.kernel(
        _deg_body,
        out_type=jax.ShapeDtypeStruct((NC, NPAD), jnp.float32),
        mesh=plsc.VectorSubcoreMesh(core_axis_name="c", subcore_axis_name="s"),
        compiler_params=pltpu.CompilerParams(needs_layout_passes=False),
        scratch_types=[
            pltpu.VMEM((3, CHUNK), jnp.int32),
            pltpu.VMEM((3, CHUNK), jnp.int32),
            pltpu.VMEM((CHUNK,), jnp.int32),
            pltpu.VMEM((CHUNK,), jnp.float32),
            pltpu.VMEM((RPT,), jnp.float32),
            pltpu.VMEM_SHARED((NPAD,), jnp.float32),
            pltpu.SemaphoreType.DMA,
            pltpu.SemaphoreType.DMA,
        ],
    )(ed)


def _agg_body(g_hbm, ed_hbm, p_hbm,
              edA, edB, rowA, rowB, colA, colB, rowsA, rowsB, acc_s,
              sgA, sgB, siA, siB, ssA, ssB):
    cid, sid, a0, nw = _worker_span()

    @pl.loop(0, CHUNK)
    def _(i):
        for j in range(D // LANES):
            rowsA[i, pl.ds(j * LANES, LANES)] = jnp.zeros((LANES,), jnp.float32)

    for k in range(RPT // CHUNK):
        pltpu.sync_copy(rowsA, acc_s.at[pl.ds(sid * RPT + k * CHUNK, CHUNK)])
    plsc.subcore_barrier()

    def _scale_fire(ed_v, col_v, rows_v, sem):
        @pl.loop(0, CHUNK, unroll=4)
        def _(e):
            s = plsc.bitcast(
                plsc.load_gather(
                    ed_v, [jnp.ones((LANES,), jnp.int32),
                           jnp.full((LANES,), e, jnp.int32)]), jnp.float32)
            for j in range(D // LANES):
                rows_v[e, pl.ds(j * LANES, LANES)] = (
                    rows_v[e, pl.ds(j * LANES, LANES)] * s)

        _unpack_lane(ed_v, 2, col_v)
        pltpu.async_copy(rows_v, acc_s.at[col_v], sem, add=True)

    pltpu.sync_copy(ed_hbm.at[a0], edA)
    _unpack_lane(edA, 0, rowA)
    pltpu.async_copy(g_hbm.at[rowA], rowsA, sgA)

    @pl.when(nw > 1)
    def _():
        pltpu.async_copy(ed_hbm.at[a0 + 1], edB, siB)

    @pl.loop(0, (nw + 1) // 2)
    def _(p):
        i = 2 * p
        pltpu.make_async_copy(g_hbm.at[rowA], rowsA, sgA).wait()

        @pl.when(i + 1 < nw)
        def _():
            pltpu.make_async_copy(ed_hbm.at[0], edB, siB).wait()

            @pl.when(p > 0)
            def _():
                pltpu.make_async_copy(rowsB, acc_s.at[colB], ssB).wait()

            _unpack_lane(edB, 0, rowB)
            pltpu.async_copy(g_hbm.at[rowB], rowsB, sgB)

        _scale_fire(edA, colA, rowsA, ssA)

        @pl.when(i + 2 < nw)
        def _():
            pltpu.async_copy(ed_hbm.at[a0 + i + 2], edA, siA)

        @pl.when(i + 1 < nw)
        def _():
            pltpu.make_async_copy(g_hbm.at[rowB], rowsB, sgB).wait()
            _scale_fire(edB, colB, rowsB, ssB)

            @pl.when(i + 3 < nw)
            def _():
                pltpu.async_copy(ed_hbm.at[a0 + i + 3], edB, siB)

        @pl.when(i + 2 < nw)
        def _():
            pltpu.make_async_copy(rowsA, acc_s.at[colA], ssA).wait()
            pltpu.make_async_copy(ed_hbm.at[0], edA, siA).wait()
            _unpack_lane(edA, 0, rowA)
            pltpu.async_copy(g_hbm.at[rowA], rowsA, sgA)

    pltpu.make_async_copy(rowsA, acc_s.at[colA], ssA).wait()

    @pl.when(nw > 1)
    def _():
        pltpu.make_async_copy(rowsB, acc_s.at[colB], ssB).wait()

    plsc.subcore_barrier()
    pltpu.sync_copy(acc_s.at[pl.ds(sid * RPT, RPT)],
                    p_hbm.at[cid, pl.ds(sid * RPT, RPT)])


def _aggregate(g, ed):
    return pl.kernel(
        _agg_body,
        out_type=jax.ShapeDtypeStruct((NC, NPAD, D), jnp.float32),
        mesh=plsc.VectorSubcoreMesh(core_axis_name="c", subcore_axis_name="s"),
        compiler_params=pltpu.CompilerParams(needs_layout_passes=False),
        scratch_types=[
            pltpu.VMEM((3, CHUNK), jnp.int32),
            pltpu.VMEM((3, CHUNK), jnp.int32),
            pltpu.VMEM((CHUNK,), jnp.int32),
            pltpu.VMEM((CHUNK,), jnp.int32),
            pltpu.VMEM((CHUNK,), jnp.int32),
            pltpu.VMEM((CHUNK,), jnp.int32),
            pltpu.VMEM((CHUNK, D), jnp.float32),
            pltpu.VMEM((CHUNK, D), jnp.float32),
            pltpu.VMEM_SHARED((NPAD, D), jnp.float32),
            pltpu.SemaphoreType.DMA,
            pltpu.SemaphoreType.DMA,
            pltpu.SemaphoreType.DMA,
            pltpu.SemaphoreType.DMA,
            pltpu.SemaphoreType.DMA,
            pltpu.SemaphoreType.DMA,
        ],
    )(g, ed)


def _mm_body(x_ref, w_ref, h_ref):
    h_ref[...] = jnp.dot(x_ref[...], w_ref[...],
                         preferred_element_type=jnp.float32)


def _matmul(x, W):
    blk = 400
    grid = N // blk
    return pl.pallas_call(
        _mm_body,
        grid=(grid,),
        in_specs=[
            pl.BlockSpec((blk, D), lambda i: (i, 0)),
            pl.BlockSpec((D, D), lambda i: (0, 0)),
        ],
        out_specs=pl.BlockSpec((blk, D), lambda i: (i, 0)),
        out_shape=jax.ShapeDtypeStruct((N, D), jnp.float32),
    )(x, W)


def _scale_body(h_ref, pd_ref, g_ref):
    deg = 1.0 + pd_ref[0] + pd_ref[1]
    dis = lax.rsqrt(deg)
    g_ref[...] = dis * h_ref[...]


def _scale_g(h, pd):
    blk = 400
    grid = N // blk
    return pl.pallas_call(
        _scale_body,
        grid=(grid,),
        in_specs=[
            pl.BlockSpec((blk, D), lambda i: (i, 0)),
            pl.BlockSpec((NC, blk, 1), lambda i: (0, i, 0)),
        ],
        out_specs=pl.BlockSpec((blk, D), lambda i: (i, 0)),
        out_shape=jax.ShapeDtypeStruct((N, D), jnp.float32),
    )(h, pd.reshape(NC, NPAD, 1))


def _fin_body(g_ref, p_ref, pd_ref, b_ref, o_ref):
    deg = 1.0 + pd_ref[0] + pd_ref[1]
    dis = lax.rsqrt(deg)
    o_ref[...] = dis * (g_ref[...] + p_ref[0] + p_ref[1]) + b_ref[...]


def _finalize(g, p, pd, b):
    blk = 400
    grid = N // blk
    return pl.pallas_call(
        _fin_body,
        grid=(grid,),
        in_specs=[
            pl.BlockSpec((blk, D), lambda i: (i, 0)),
            pl.BlockSpec((NC, blk, D), lambda i: (0, i, 0)),
            pl.BlockSpec((NC, blk, 1), lambda i: (0, i, 0)),
            pl.BlockSpec((1, D), lambda i: (0, 0)),
        ],
        out_specs=pl.BlockSpec((blk, D), lambda i: (i, 0)),
        out_shape=jax.ShapeDtypeStruct((N, D), jnp.float32),
    )(g, p, pd.reshape(NC, NPAD, 1), b.reshape(1, D))


def kernel(x, edge_index, edge_weight, W, b):
    row = edge_index[0]
    col = edge_index[1]
    ed = jnp.stack(
        [row.reshape(NCHUNK, CHUNK),
         lax.bitcast_convert_type(edge_weight, jnp.int32).reshape(NCHUNK, CHUNK),
         col.reshape(NCHUNK, CHUNK)], axis=1)
    h = _matmul(x, W)
    pd = _deg_partials(ed)
    g = _scale_g(h, pd)
    p = _aggregate(g, ed)
    return _finalize(g, p, pd, b)


# speedup vs baseline: 1.0344x; 1.0344x over previous
import jax
import jax.numpy as jnp
from jax import lax
from jax.experimental import pallas as pl
from jax.experimental.pallas import tpu as pltpu
from jax.experimental.pallas import tpu_sc as plsc

N = 10000
E = 320000
D = 128

NC = 2
NS = 16
NW = NC * NS
LANES = 16

CHUNK = 128
NCHUNK = E // CHUNK
BASE = NCHUNK // NW
REM = NCHUNK % NW
RPT = 640
NPAD = NS * RPT


def _worker_span():
    cid = lax.axis_index("c")
    sid = lax.axis_index("s")
    wid = sid * NC + cid
    nw = BASE + jnp.where(wid < REM, 1, 0)
    a0 = wid * BASE + jnp.minimum(wid, REM)
    return cid, sid, a0, nw


def _deg_body(ei_hbm, ew_hbm, pd_hbm, colA, colB, ewA, ewB, zb_v, deg_s,
              siA, siB):
    cid, sid, a0, nw = _worker_span()

    def _fire(c_idx, col_v, ew_v, sem):
        pltpu.async_copy(ei_hbm.at[1, pl.ds(c_idx * CHUNK, CHUNK)], col_v, sem)
        pltpu.async_copy(ew_hbm.at[pl.ds(c_idx * CHUNK, CHUNK)], ew_v, sem)

    def _wait(col_v, ew_v, sem):
        pltpu.make_async_copy(ei_hbm.at[1, pl.ds(0, CHUNK)], col_v, sem).wait()
        pltpu.make_async_copy(ew_hbm.at[pl.ds(0, CHUNK)], ew_v, sem).wait()

    @pl.loop(0, RPT // LANES)
    def _(i):
        zb_v[pl.ds(i * LANES, LANES)] = jnp.zeros((LANES,), jnp.float32)

    pltpu.sync_copy(zb_v, deg_s.at[pl.ds(sid * RPT, RPT)])
    plsc.subcore_barrier()

    _fire(a0, colA, ewA, siA)

    @pl.when(nw > 1)
    def _():
        _fire(a0 + 1, colB, ewB, siB)

    @pl.loop(0, (nw + 1) // 2)
    def _(p):
        i = 2 * p
        _wait(colA, ewA, siA)
        pltpu.sync_copy(ewA, deg_s.at[colA], add=True)

        @pl.when(i + 2 < nw)
        def _():
            _fire(a0 + i + 2, colA, ewA, siA)

        @pl.when(i + 1 < nw)
        def _():
            _wait(colB, ewB, siB)
            pltpu.sync_copy(ewB, deg_s.at[colB], add=True)

            @pl.when(i + 3 < nw)
            def _():
                _fire(a0 + i + 3, colB, ewB, siB)

    plsc.subcore_barrier()
    pltpu.sync_copy(deg_s.at[pl.ds(sid * RPT, RPT)],
                    pd_hbm.at[cid, pl.ds(sid * RPT, RPT)])


def _deg_partials(ei, ew):
    return pl.kernel(
        _deg_body,
        out_type=jax.ShapeDtypeStruct((NC, NPAD), jnp.float32),
        mesh=plsc.VectorSubcoreMesh(core_axis_name="c", subcore_axis_name="s"),
        compiler_params=pltpu.CompilerParams(needs_layout_passes=False),
        scratch_types=[
            pltpu.VMEM((CHUNK,), jnp.int32),
            pltpu.VMEM((CHUNK,), jnp.int32),
            pltpu.VMEM((CHUNK,), jnp.float32),
            pltpu.VMEM((CHUNK,), jnp.float32),
            pltpu.VMEM((RPT,), jnp.float32),
            pltpu.VMEM_SHARED((NPAD,), jnp.float32),
            pltpu.SemaphoreType.DMA,
            pltpu.SemaphoreType.DMA,
        ],
    )(ei, ew)


def _agg_body(g_hbm, ei_hbm, ew_hbm, p_hbm,
              rowA, rowB, colA, colB, ewA, ewB, rowsA, rowsB, acc_s,
              sgA, sgB, siA, siB, scA, scB, ssA, ssB):
    cid, sid, a0, nw = _worker_span()

    def _fire_re(c_idx, row_v, ew_v, sem):
        pltpu.async_copy(ei_hbm.at[0, pl.ds(c_idx * CHUNK, CHUNK)], row_v, sem)
        pltpu.async_copy(ew_hbm.at[pl.ds(c_idx * CHUNK, CHUNK)], ew_v, sem)

    def _wait_re(row_v, ew_v, sem):
        pltpu.make_async_copy(ei_hbm.at[0, pl.ds(0, CHUNK)], row_v, sem).wait()
        pltpu.make_async_copy(ew_hbm.at[pl.ds(0, CHUNK)], ew_v, sem).wait()

    def _fire_col(c_idx, col_v, sem):
        pltpu.async_copy(ei_hbm.at[1, pl.ds(c_idx * CHUNK, CHUNK)], col_v, sem)

    @pl.loop(0, CHUNK)
    def _(i):
        for j in range(D // LANES):
            rowsA[i, pl.ds(j * LANES, LANES)] = jnp.zeros((LANES,), jnp.float32)

    for k in range(RPT // CHUNK):
        pltpu.sync_copy(rowsA, acc_s.at[pl.ds(sid * RPT + k * CHUNK, CHUNK)])
    plsc.subcore_barrier()

    def _scale_fire(ew_v, col_v, rows_v, csem, ssem):
        @pl.loop(0, CHUNK, unroll=4)
        def _(e):
            s = plsc.load_gather(ew_v, [jnp.full((LANES,), e, jnp.int32)])
            for j in range(D // LANES):
                rows_v[e, pl.ds(j * LANES, LANES)] = (
                    rows_v[e, pl.ds(j * LANES, LANES)] * s)

        pltpu.make_async_copy(ei_hbm.at[1, pl.ds(0, CHUNK)], col_v,
                              csem).wait()
        pltpu.async_copy(rows_v, acc_s.at[col_v], ssem, add=True)

    _fire_re(a0, rowA, ewA, siA)
    _fire_col(a0, colA, scA)
    _wait_re(rowA, ewA, siA)
    pltpu.async_copy(g_hbm.at[rowA], rowsA, sgA)

    @pl.when(nw > 1)
    def _():
        _fire_re(a0 + 1, rowB, ewB, siB)

    @pl.loop(0, (nw + 1) // 2)
    def _(p):
        i = 2 * p
        pltpu.make_async_copy(g_hbm.at[rowA], rowsA, sgA).wait()

        @pl.when(i + 1 < nw)
        def _():
            _wait_re(rowB, ewB, siB)

            @pl.when(p > 0)
            def _():
                pltpu.make_async_copy(rowsB, acc_s.at[colB], ssB).wait()

            _fire_col(a0 + i + 1, colB, scB)
            pltpu.async_copy(g_hbm.at[rowB], rowsB, sgB)

        _scale_fire(ewA, colA, rowsA, scA, ssA)

        @pl.when(i + 2 < nw)
        def _():
            _fire_re(a0 + i + 2, rowA, ewA, siA)

        @pl.when(i + 1 < nw)
        def _():
            pltpu.make_async_copy(g_hbm.at[rowB], rowsB, sgB).wait()
            _scale_fire(ewB, colB, rowsB, scB, ssB)

            @pl.when(i + 3 < nw)
            def _():
                _fire_re(a0 + i + 3, rowB, ewB, siB)

        @pl.when(i + 2 < nw)
        def _():
            pltpu.make_async_copy(rowsA, acc_s.at[colA], ssA).wait()
            _fire_col(a0 + i + 2, colA, scA)
            _wait_re(rowA, ewA, siA)
            pltpu.async_copy(g_hbm.at[rowA], rowsA, sgA)

    pltpu.make_async_copy(rowsA, acc_s.at[colA], ssA).wait()

    @pl.when(nw > 1)
    def _():
        pltpu.make_async_copy(rowsB, acc_s.at[colB], ssB).wait()

    plsc.subcore_barrier()
    pltpu.sync_copy(acc_s.at[pl.ds(sid * RPT, RPT)],
                    p_hbm.at[cid, pl.ds(sid * RPT, RPT)])


def _aggregate(g, ei, ew):
    return pl.kernel(
        _agg_body,
        out_type=jax.ShapeDtypeStruct((NC, NPAD, D), jnp.float32),
        mesh=plsc.VectorSubcoreMesh(core_axis_name="c", subcore_axis_name="s"),
        compiler_params=pltpu.CompilerParams(needs_layout_passes=False),
        scratch_types=[
            pltpu.VMEM((CHUNK,), jnp.int32),
            pltpu.VMEM((CHUNK,), jnp.int32),
            pltpu.VMEM((CHUNK,), jnp.int32),
            pltpu.VMEM((CHUNK,), jnp.int32),
            pltpu.VMEM((CHUNK,), jnp.float32),
            pltpu.VMEM((CHUNK,), jnp.float32),
            pltpu.VMEM((CHUNK, D), jnp.float32),
            pltpu.VMEM((CHUNK, D), jnp.float32),
            pltpu.VMEM_SHARED((NPAD, D), jnp.float32),
            pltpu.SemaphoreType.DMA,
            pltpu.SemaphoreType.DMA,
            pltpu.SemaphoreType.DMA,
            pltpu.SemaphoreType.DMA,
            pltpu.SemaphoreType.DMA,
            pltpu.SemaphoreType.DMA,
            pltpu.SemaphoreType.DMA,
            pltpu.SemaphoreType.DMA,
        ],
    )(g, ei, ew)


def _mm_body(x_ref, w_ref, h_ref):
    h_ref[...] = jnp.dot(x_ref[...], w_ref[...],
                         preferred_element_type=jnp.float32)


def _matmul(x, W):
    blk = 400
    grid = N // blk
    return pl.pallas_call(
        _mm_body,
        grid=(grid,),
        in_specs=[
            pl.BlockSpec((blk, D), lambda i: (i, 0)),
            pl.BlockSpec((D, D), lambda i: (0, 0)),
        ],
        out_specs=pl.BlockSpec((blk, D), lambda i: (i, 0)),
        out_shape=jax.ShapeDtypeStruct((N, D), jnp.float32),
    )(x, W)


def _scale_body(h_ref, pd_ref, g_ref):
    deg = 1.0 + pd_ref[0] + pd_ref[1]
    dis = lax.rsqrt(deg)
    g_ref[...] = dis * h_ref[...]


def _scale_g(h, pd):
    blk = 400
    grid = N // blk
    return pl.pallas_call(
        _scale_body,
        grid=(grid,),
        in_specs=[
            pl.BlockSpec((blk, D), lambda i: (i, 0)),
            pl.BlockSpec((NC, blk, 1), lambda i: (0, i, 0)),
        ],
        out_specs=pl.BlockSpec((blk, D), lambda i: (i, 0)),
        out_shape=jax.ShapeDtypeStruct((N, D), jnp.float32),
    )(h, pd.reshape(NC, NPAD, 1))


def _fin_body(g_ref, p_ref, pd_ref, b_ref, o_ref):
    deg = 1.0 + pd_ref[0] + pd_ref[1]
    dis = lax.rsqrt(deg)
    o_ref[...] = dis * (g_ref[...] + p_ref[0] + p_ref[1]) + b_ref[...]


def _finalize(g, p, pd, b):
    blk = 400
    grid = N // blk
    return pl.pallas_call(
        _fin_body,
        grid=(grid,),
        in_specs=[
            pl.BlockSpec((blk, D), lambda i: (i, 0)),
            pl.BlockSpec((NC, blk, D), lambda i: (0, i, 0)),
            pl.BlockSpec((NC, blk, 1), lambda i: (0, i, 0)),
            pl.BlockSpec((1, D), lambda i: (0, 0)),
        ],
        out_specs=pl.BlockSpec((blk, D), lambda i: (i, 0)),
        out_shape=jax.ShapeDtypeStruct((N, D), jnp.float32),
    )(g, p, pd.reshape(NC, NPAD, 1), b.reshape(1, D))


def kernel(x, edge_index, edge_weight, W, b):
    h = _matmul(x, W)
    pd = _deg_partials(edge_index, edge_weight)
    g = _scale_g(h, pd)
    p = _aggregate(g, edge_index, edge_weight)
    return _finalize(g, p, pd, b)
